# baseline (speedup 1.0000x reference)
import jax
import jax.numpy as jnp
from jax import lax
from jax.experimental import pallas as pl
from jax.experimental.pallas import tpu as pltpu
from jax.experimental.pallas import tpu_sc as plsc

_B = 4
_L = 8192
_D = 1024
_BLK = 256

_info = plsc.get_sparse_core_info()
_NC = _info.num_cores
_NS = _info.num_subcores
_NW = _NC * _NS
_ROWS_PER_W = _L // _NW
_CHUNK = 64
_NCHUNK = _ROWS_PER_W // _CHUNK

_B_TC = 2
_SC_BATCHES = (2, 3)


def _sc_body(table_hbm, out_hbm, buf, sem):
    wid = lax.axis_index("s") * _NC + lax.axis_index("c")
    base = wid * _ROWS_PER_W
    for i in range(_NCHUNK):
        row0 = base + i * _CHUNK
        pltpu.async_copy(table_hbm.at[pl.ds(row0, _CHUNK)], buf, sem).wait()
        for b in _SC_BATCHES:
            pltpu.async_copy(buf, out_hbm.at[b, pl.ds(row0, _CHUNK)], sem).wait()


def _tc_body(table_ref, partial_ref, out_ref):
    del partial_ref
    out_ref[...] = jnp.broadcast_to(table_ref[...][None], (_B_TC, _BLK, _D))


def kernel(x, table):
    del x
    mesh = plsc.VectorSubcoreMesh(core_axis_name="c", subcore_axis_name="s")
    sc_run = pl.kernel(
        _sc_body,
        mesh=mesh,
        out_type=jax.ShapeDtypeStruct((_B, _L, _D), jnp.float32),
        scratch_types=[
            pltpu.VMEM((_CHUNK, _D), jnp.float32),
            pltpu.SemaphoreType.DMA,
        ],
    )
    partial = sc_run(table)
    return pl.pallas_call(
        _tc_body,
        grid=(_L // _BLK,),
        in_specs=[
            pl.BlockSpec((_BLK, _D), lambda j: (j, 0)),
            pl.BlockSpec(memory_space=pl.ANY),
        ],
        out_specs=pl.BlockSpec((_B_TC, _BLK, _D), lambda j: (0, j, 0)),
        out_shape=jax.ShapeDtypeStruct((_B, _L, _D), jnp.float32),
        input_output_aliases={1: 0},
    )(table, partial)


# speedup vs baseline: 1.2434x; 1.2434x over previous
import jax
import jax.numpy as jnp
from jax import lax
from jax.experimental import pallas as pl
from jax.experimental.pallas import tpu as pltpu
from jax.experimental.pallas import tpu_sc as plsc

_B = 4
_L = 8192
_D = 1024

_info = plsc.get_sparse_core_info()
_NC = _info.num_cores
_NS = _info.num_subcores
_NW = _NC * _NS
_ROWS_PER_W = _L // _NW
_CHUNK = 32
_NCHUNK = _ROWS_PER_W // _CHUNK
_NBUF = 3


def _copy_body(table_hbm, out_hbm, buf0, buf1, buf2,
               ld0, ld1, ld2, st0, st1, st2):
    wid = lax.axis_index("s") * _NC + lax.axis_index("c")
    base = wid * _ROWS_PER_W
    bufs = (buf0, buf1, buf2)
    ld_sems = (ld0, ld1, ld2)
    st_sems = (st0, st1, st2)
    loads = [None] * _NCHUNK
    stores = [[] for _ in range(_NBUF)]
    for i in range(2):
        loads[i] = pltpu.async_copy(
            table_hbm.at[pl.ds(base + i * _CHUNK, _CHUNK)], bufs[i], ld_sems[i]
        )
    for i in range(_NCHUNK):
        k = i % _NBUF
        loads[i].wait()
        row0 = base + i * _CHUNK
        stores[k] = [
            pltpu.async_copy(bufs[k], out_hbm.at[b, pl.ds(row0, _CHUNK)], st_sems[k])
            for b in range(_B)
        ]
        nxt = i + 2
        if nxt < _NCHUNK:
            kk = nxt % _NBUF
            for c in stores[kk]:
                c.wait()
            stores[kk] = []
            loads[nxt] = pltpu.async_copy(
                table_hbm.at[pl.ds(base + nxt * _CHUNK, _CHUNK)], bufs[kk], ld_sems[kk]
            )
    for lst in stores:
        for c in lst:
            c.wait()


def kernel(x, table):
    del x
    mesh = plsc.VectorSubcoreMesh(core_axis_name="c", subcore_axis_name="s")
    run = pl.kernel(
        _copy_body,
        mesh=mesh,
        out_type=jax.ShapeDtypeStruct((_B, _L, _D), jnp.float32),
        scratch_types=[
            pltpu.VMEM((_CHUNK, _D), jnp.float32),
            pltpu.VMEM((_CHUNK, _D), jnp.float32),
            pltpu.VMEM((_CHUNK, _D), jnp.float32),
            pltpu.SemaphoreType.DMA,
            pltpu.SemaphoreType.DMA,
            pltpu.SemaphoreType.DMA,
            pltpu.SemaphoreType.DMA,
            pltpu.SemaphoreType.DMA,
            pltpu.SemaphoreType.DMA,
        ],
    )
    return run(table)


# speedup vs baseline: 1.2950x; 1.0415x over previous
import jax
import jax.numpy as jnp
from jax import lax
from jax.experimental import pallas as pl
from jax.experimental.pallas import tpu as pltpu
from jax.experimental.pallas import tpu_sc as plsc

_B = 4
_L = 8192
_D = 1024

_info = plsc.get_sparse_core_info()
_NC = _info.num_cores
_NS = _info.num_subcores
_NW = _NC * _NS
_ROWS_PER_W = _L // _NW
_CHUNKS = (88, 88, 80)
_MAXCHUNK = max(_CHUNKS)


def _copy_body(table_hbm, out_hbm, buf, sem):
    wid = lax.axis_index("s") * _NC + lax.axis_index("c")
    base = wid * _ROWS_PER_W
    off = 0
    for rows in _CHUNKS:
        row0 = base + off
        pltpu.async_copy(
            table_hbm.at[pl.ds(row0, rows)], buf.at[pl.ds(0, rows)], sem
        ).wait()
        for b in range(_B):
            pltpu.async_copy(
                buf.at[pl.ds(0, rows)], out_hbm.at[b, pl.ds(row0, rows)], sem
            ).wait()
        off += rows


def kernel(x, table):
    del x
    mesh = plsc.VectorSubcoreMesh(core_axis_name="c", subcore_axis_name="s")
    run = pl.kernel(
        _copy_body,
        mesh=mesh,
        out_type=jax.ShapeDtypeStruct((_B, _L, _D), jnp.float32),
        scratch_types=[
            pltpu.VMEM((_MAXCHUNK, _D), jnp.float32),
            pltpu.SemaphoreType.DMA,
        ],
    )
    return run(table)


# speedup vs baseline: 1.2962x; 1.0010x over previous
import jax
import jax.numpy as jnp
from jax import lax
from jax.experimental import pallas as pl
from jax.experimental.pallas import tpu as pltpu
from jax.experimental.pallas import tpu_sc as plsc

_B = 4
_L = 8192
_D = 1024

_info = plsc.get_sparse_core_info()
_NC = _info.num_cores
_NS = _info.num_subcores
_NW = _NC * _NS
_ROWS_PER_W = _L // _NW
_CHUNKS = (120, 120, 16)
_MAXCHUNK = max(_CHUNKS)


def _copy_body(table_hbm, out_hbm, buf, sem):
    wid = lax.axis_index("s") * _NC + lax.axis_index("c")
    base = wid * _ROWS_PER_W
    off = 0
    for rows in _CHUNKS:
        row0 = base + off
        pltpu.async_copy(
            table_hbm.at[pl.ds(row0, rows)], buf.at[pl.ds(0, rows)], sem
        ).wait()
        for b in range(_B):
            pltpu.async_copy(
                buf.at[pl.ds(0, rows)], out_hbm.at[b, pl.ds(row0, rows)], sem
            ).wait()
        off += rows


def kernel(x, table):
    del x
    mesh = plsc.VectorSubcoreMesh(core_axis_name="c", subcore_axis_name="s")
    run = pl.kernel(
        _copy_body,
        mesh=mesh,
        out_type=jax.ShapeDtypeStruct((_B, _L, _D), jnp.float32),
        scratch_types=[
            pltpu.VMEM((_MAXCHUNK, _D), jnp.float32),
            pltpu.SemaphoreType.DMA,
        ],
    )
    return run(table)
